# baseline (speedup 1.0000x reference)
import functools

import jax
import jax.numpy as jnp
from jax import lax
from jax.experimental import pallas as pl
from jax.experimental.pallas import tpu as pltpu
from jax.experimental.pallas import tpu_sc as plsc


def _make_gather(B, D, C, interpret=False):
    NC, NS = 2, 16
    NW = NC * NS
    assert B % NW == 0
    b_per_w = B // NW
    assert b_per_w % C == 0
    n_chunks = b_per_w // C
    mesh = plsc.VectorSubcoreMesh(
        core_axis_name="c", subcore_axis_name="s", num_cores=NC, num_subcores=NS
    )

    @functools.partial(
        pl.kernel,
        out_type=jax.ShapeDtypeStruct((B, D), jnp.float32),
        mesh=mesh,
        scratch_types=[
            pltpu.VMEM((b_per_w,), jnp.int32),
            pltpu.VMEM((C, D), jnp.float32),
            pltpu.SemaphoreType.DMA,
        ],
        interpret=interpret,
        compiler_params=pltpu.CompilerParams(use_tc_tiling_on_sc=False),
    )
    def k(idx_hbm, table_hbm, out_hbm, idx_v, rows_v, sem):
        wid = lax.axis_index("s") * NC + lax.axis_index("c")
        base = wid * b_per_w
        pltpu.sync_copy(idx_hbm.at[pl.ds(base, b_per_w)], idx_v)

        def body(j, carry):
            off = j * C
            pltpu.async_copy(
                table_hbm.at[idx_v.at[pl.ds(off, C)]], rows_v, sem
            ).wait()
            pltpu.sync_copy(rows_v, out_hbm.at[pl.ds(base + off, C)])
            return carry

        lax.fori_loop(0, n_chunks, body, 0)

    return k


def kernel(x, vertex_id, dim):
    idx = (vertex_id + dim).astype(jnp.int32)
    B = idx.shape[0]
    D = x.shape[1]
    return _make_gather(B, D, C=128)(idx, x)


# speedup vs baseline: 1.0795x; 1.0795x over previous
import functools

import jax
import jax.numpy as jnp
from jax import lax
from jax.experimental import pallas as pl
from jax.experimental.pallas import tpu as pltpu
from jax.experimental.pallas import tpu_sc as plsc


def _make_gather(B, D, C, nbuf=2, interpret=False):
    NC, NS = 2, 16
    NW = NC * NS
    assert B % NW == 0
    b_per_w = B // NW
    assert b_per_w % C == 0
    n_chunks = b_per_w // C
    assert n_chunks >= nbuf
    mesh = plsc.VectorSubcoreMesh(
        core_axis_name="c", subcore_axis_name="s", num_cores=NC, num_subcores=NS
    )

    @functools.partial(
        pl.kernel,
        out_type=jax.ShapeDtypeStruct((B, D), jnp.float32),
        mesh=mesh,
        scratch_types=[
            pltpu.VMEM((b_per_w,), jnp.int32),
            pltpu.VMEM((nbuf, C, D), jnp.float32),
            [pltpu.SemaphoreType.DMA] * nbuf,
            [pltpu.SemaphoreType.DMA] * nbuf,
        ],
        interpret=interpret,
        compiler_params=pltpu.CompilerParams(use_tc_tiling_on_sc=False),
    )
    def k(idx_hbm, table_hbm, out_hbm, idx_v, rows_v, gsems, wsems):
        wid = lax.axis_index("s") * NC + lax.axis_index("c")
        base = wid * b_per_w
        pltpu.sync_copy(idx_hbm.at[pl.ds(base, b_per_w)], idx_v)

        def start_gather(g):
            b = g % nbuf
            return pltpu.async_copy(
                table_hbm.at[idx_v.at[pl.ds(g * C, C)]], rows_v.at[b], gsems[b]
            )

        def start_write(g):
            b = g % nbuf
            return pltpu.async_copy(
                rows_v.at[b], out_hbm.at[pl.ds(base + g * C, C)], wsems[b]
            )

        gcopies = [None] * n_chunks
        wcopies = [None] * n_chunks
        for b in range(nbuf):
            gcopies[b] = start_gather(b)
        for g in range(n_chunks):
            gcopies[g].wait()
            wcopies[g] = start_write(g)
            gn = g + nbuf
            if gn < n_chunks:
                wcopies[g].wait()
                gcopies[gn] = start_gather(gn)
        for g in range(n_chunks - nbuf, n_chunks):
            if g >= 0 and wcopies[g] is not None:
                wcopies[g].wait()

    return k


def kernel(x, vertex_id, dim):
    idx = (vertex_id + dim).astype(jnp.int32)
    B = idx.shape[0]
    D = x.shape[1]
    return _make_gather(B, D, C=832, nbuf=2)(idx, x)


# speedup vs baseline: 1.5338x; 1.4208x over previous
import functools

import jax
import jax.numpy as jnp
from jax import lax
from jax.experimental import pallas as pl
from jax.experimental.pallas import tpu as pltpu
from jax.experimental.pallas import tpu_sc as plsc


def _make_gather(B, D, C, nbuf=2, interpret=False):
    NC, NS = 2, 16
    NW = NC * NS
    assert B % NW == 0
    b_per_w = B // NW
    assert b_per_w % C == 0
    n_chunks = b_per_w // C
    assert n_chunks >= nbuf
    mesh = plsc.VectorSubcoreMesh(
        core_axis_name="c", subcore_axis_name="s", num_cores=NC, num_subcores=NS
    )

    @functools.partial(
        pl.kernel,
        out_type=jax.ShapeDtypeStruct((B, D), jnp.float32),
        mesh=mesh,
        scratch_types=[
            [pltpu.SMEM((C,), jnp.int32)] * nbuf,
            pltpu.VMEM((b_per_w,), jnp.int32),
            pltpu.VMEM((nbuf, C, D), jnp.float32),
            [pltpu.SemaphoreType.DMA] * nbuf,
            [pltpu.SemaphoreType.DMA] * nbuf,
        ],
        interpret=interpret,
    )
    def k(idx_hbm, table_hbm, out_hbm, idx_s, idx_v, rows_v, gsems, wsems):
        wid = lax.axis_index("s") * NC + lax.axis_index("c")
        base = pl.multiple_of(wid * b_per_w, b_per_w)
        pltpu.sync_copy(idx_hbm.at[pl.ds(base, b_per_w)], idx_v)

        def start_gather(g):
            b = g % nbuf

            def row16(v, carry):
                j0 = v * 16
                vec = idx_v[pl.ds(g * C + j0, 16)]
                for l in range(16):
                    pltpu.async_copy(
                        table_hbm.at[pl.ds(vec[l], 1), :],
                        rows_v.at[b].at[pl.ds(j0 + l, 1), :],
                        gsems[b],
                    )
                return carry

            lax.fori_loop(0, C // 16, row16, 0)

        def wait_gather(g):
            b = g % nbuf
            pltpu.make_async_copy(
                table_hbm.at[pl.ds(0, C), :], rows_v.at[b], gsems[b]
            ).wait()

        def start_write(g):
            b = g % nbuf
            return pltpu.async_copy(
                rows_v.at[b], out_hbm.at[pl.ds(base + g * C, C), :], wsems[b]
            )

        wcopies = [None] * n_chunks
        for g in range(n_chunks):
            if g >= nbuf:
                wcopies[g - nbuf].wait()
            start_gather(g)
            wait_gather(g)
            wcopies[g] = start_write(g)
        for g in range(max(0, n_chunks - nbuf), n_chunks):
            if wcopies[g] is not None:
                wcopies[g].wait()

    return k


def kernel(x, vertex_id, dim):
    idx = (vertex_id + dim).astype(jnp.int32)
    B = idx.shape[0]
    D = x.shape[1]
    return _make_gather(B, D, C=256, nbuf=2)(idx, x)


# speedup vs baseline: 1.5596x; 1.0168x over previous
import functools

import jax
import jax.numpy as jnp
from jax import lax
from jax.experimental import pallas as pl
from jax.experimental.pallas import tpu as pltpu
from jax.experimental.pallas import tpu_sc as plsc


def _make_gather(B, D, C, nbuf=2, interpret=False):
    NC, NS = 2, 16
    NW = NC * NS
    assert B % NW == 0
    b_per_w = B // NW
    assert b_per_w % C == 0
    n_chunks = b_per_w // C
    assert n_chunks >= nbuf
    mesh = plsc.VectorSubcoreMesh(
        core_axis_name="c", subcore_axis_name="s", num_cores=NC, num_subcores=NS
    )

    @functools.partial(
        pl.kernel,
        out_type=jax.ShapeDtypeStruct((B, D), jnp.float32),
        mesh=mesh,
        scratch_types=[
            [pltpu.SMEM((C,), jnp.int32)] * nbuf,
            pltpu.VMEM((b_per_w,), jnp.int32),
            pltpu.VMEM((nbuf, C, D), jnp.float32),
            [pltpu.SemaphoreType.DMA] * nbuf,
            [pltpu.SemaphoreType.DMA] * nbuf,
        ],
        interpret=interpret,
    )
    def k(idx_hbm, table_hbm, out_hbm, idx_s, idx_v, rows_v, gsems, wsems):
        wid = lax.axis_index("s") * NC + lax.axis_index("c")
        base = pl.multiple_of(wid * b_per_w, b_per_w)
        pltpu.sync_copy(idx_hbm.at[pl.ds(base, b_per_w)], idx_v)

        def start_gather(g):
            b = g % nbuf

            def row16(v, carry):
                j0 = v * 16
                vec = idx_v[pl.ds(g * C + j0, 16)]
                for l in range(16):
                    pltpu.async_copy(
                        table_hbm.at[pl.ds(vec[l], 1), :],
                        rows_v.at[b].at[pl.ds(j0 + l, 1), :],
                        gsems[b],
                    )
                return carry

            lax.fori_loop(0, C // 16, row16, 0)

        def wait_gather(g):
            b = g % nbuf
            pltpu.make_async_copy(
                table_hbm.at[pl.ds(0, C), :], rows_v.at[b], gsems[b]
            ).wait()

        def start_write(g):
            b = g % nbuf
            return pltpu.async_copy(
                rows_v.at[b], out_hbm.at[pl.ds(base + g * C, C), :], wsems[b]
            )

        wcopies = [None] * n_chunks
        start_gather(0)
        for g in range(n_chunks):
            gn = g + 1
            if gn < n_chunks:
                if gn >= nbuf:
                    wcopies[gn - nbuf].wait()
                start_gather(gn)
            wait_gather(g)
            wcopies[g] = start_write(g)
        for g in range(max(0, n_chunks - nbuf), n_chunks):
            if wcopies[g] is not None:
                wcopies[g].wait()

    return k


def kernel(x, vertex_id, dim):
    idx = (vertex_id + dim).astype(jnp.int32)
    B = idx.shape[0]
    D = x.shape[1]
    return _make_gather(B, D, C=416, nbuf=2)(idx, x)
